# Initial kernel scaffold; baseline (speedup 1.0000x reference)
import jax
import jax.numpy as jnp
from jax.experimental import pallas as pl


def kernel(h_init, edge_index, W1, Wa):
    raise NotImplementedError("write your pallas kernel here")


# speedup vs baseline: 15.3294x; 15.3294x over previous
import functools

import jax
import jax.numpy as jnp
from jax import lax
from jax.experimental import pallas as pl
from jax.experimental.pallas import tpu as pltpu
from jax.experimental.pallas import tpu_sc as plsc

N = 10000
E = 320000
D = 128

NC = 2
NS = 16
NW = NC * NS
EPW = E // NW
K = 80
NG = K // 16
NCHUNK = EPW // K
NP = 10240
ROWS_PT = NP // NS
L = 16
TRASH = N + 100


def _tc_fwd(h_init, W1T, WaT):
    B = 1000

    def body(hi_ref, w1_ref, wa_ref, h_ref, p_ref):
        h = jnp.dot(hi_ref[...], w1_ref[...], preferred_element_type=jnp.float32)
        h_ref[...] = h
        p_ref[...] = jnp.dot(h, wa_ref[...], preferred_element_type=jnp.float32)

    return pl.pallas_call(
        body,
        grid=(N // B,),
        in_specs=[
            pl.BlockSpec((B, D), lambda i: (i, 0)),
            pl.BlockSpec((D, D), lambda i: (0, 0)),
            pl.BlockSpec((D, 1), lambda i: (0, 0)),
        ],
        out_specs=[
            pl.BlockSpec((B, D), lambda i: (i, 0)),
            pl.BlockSpec((B, 1), lambda i: (i, 0)),
        ],
        out_shape=[
            jax.ShapeDtypeStruct((N, D), jnp.float32),
            jax.ShapeDtypeStruct((N, 1), jnp.float32),
        ],
    )(h_init, W1T, WaT)


_SC_MESH = plsc.VectorSubcoreMesh(core_axis_name="c", subcore_axis_name="s")


@functools.partial(
    pl.kernel,
    out_type=[
        jax.ShapeDtypeStruct((NC, NP, D), jnp.float32),
        jax.ShapeDtypeStruct((NW * N,), jnp.float32),
    ],
    mesh=_SC_MESH,
    compiler_params=pltpu.CompilerParams(needs_layout_passes=False),
    scratch_types=[
        pltpu.VMEM((N,), jnp.float32),
        pltpu.VMEM((NP,), jnp.float32),
        pltpu.VMEM((N,), jnp.int32),
        pltpu.VMEM((K,), jnp.int32),
        pltpu.VMEM((K,), jnp.int32),
        pltpu.VMEM((K,), jnp.float32),
        pltpu.VMEM((K, D), jnp.float32),
        pltpu.VMEM_SHARED((NP, D), jnp.float32),
        pltpu.SemaphoreType.DMA,
    ],
)
def _sc_edges(h_hbm, p_hbm, src_hbm, dst_hbm, u_out, d_out,
              p_v, den_v, tag_v, sidx_v, didx_v, uex_v, rows_v, u_sh, sem):
    core = lax.axis_index("c")
    sub = lax.axis_index("s")
    g = core * NS + sub
    base = g * EPW
    tb = sub * ROWS_PT

    zero16 = jnp.zeros((L,), jnp.float32)
    izero16 = jnp.zeros((L,), jnp.int32)
    iota16 = lax.iota(jnp.int32, L)

    cbase_v, _ = plsc.scan_count(iota16)
    cbase = cbase_v[0]

    pltpu.sync_copy(p_hbm, p_v)

    def _zden(i, _):
        den_v[pl.ds(i * L, L)] = zero16
        return ()
    lax.fori_loop(0, NP // L, _zden, ())

    def _ztag(i, _):
        tag_v[pl.ds(i * L, L)] = izero16
        return ()
    lax.fori_loop(0, N // L, _ztag, ())

    def _zrows(i, _):
        for j in range(D // L):
            rows_v[i, pl.ds(j * L, L)] = zero16
        return ()
    lax.fori_loop(0, K, _zrows, ())

    for k in range(ROWS_PT // K):
        pltpu.sync_copy(rows_v, u_sh.at[pl.ds(tb + k * K, K)])
    plsc.subcore_barrier()

    def chunk_body(ci, _):
        off = base + ci * K
        epoch = (ci + 1) * 128
        pltpu.sync_copy(src_hbm.at[pl.ds(off, K)], sidx_v)
        pltpu.sync_copy(dst_hbm.at[pl.ds(off, K)], didx_v)
        pltpu.async_copy(h_hbm.at[sidx_v], rows_v, sem).wait()

        uvals = []
        for gi in range(NG):
            s16 = sidx_v[pl.ds(gi * L, L)]
            d16 = didx_v[pl.ds(gi * L, L)]
            ps = plsc.load_gather(p_v, [s16])
            pd = plsc.load_gather(p_v, [d16])
            z = jnp.exp((pd - ps) * 2.0)
            u = jnp.exp(1.0 - 2.0 / (z + 1.0))
            uex_v[pl.ds(gi * L, L)] = u
            uvals.append(u)

        def _scale(gi, _):
            a16 = uex_v[pl.ds(gi * L, L)]
            eb = gi * L
            for l in range(L):
                a = a16[l]
                for j in range(D // L):
                    sl = pl.ds(j * L, L)
                    rows_v[eb + l, sl] = rows_v[eb + l, sl] * a
            return ()
        lax.fori_loop(0, NG, _scale, ())

        for gi in range(NG):
            d16 = didx_v[pl.ds(gi * L, L)]
            pos16 = iota16 + (gi * L)
            cnt, _last = plsc.scan_count(d16)
            m_first = cnt == cbase
            t = plsc.load_gather(tag_v, [d16])
            m_cross = t >= epoch
            m_ins = jnp.logical_and(m_first, jnp.logical_not(m_cross))
            plsc.store_scatter(tag_v, [d16], epoch + pos16, mask=m_ins)
            t2 = plsc.load_gather(tag_v, [d16])
            kp16 = t2 - epoch
            m_comb = kp16 != pos16
            ndup = plsc.all_reduce_population_count(m_comb)[0]

            mi16 = m_comb.astype(jnp.int32)

            @pl.when(ndup > 0)
            def _fixup(gi=gi, kp16=kp16, mi16=mi16):
                for l in range(L):
                    kp = kp16[l]

                    @pl.when(mi16[l] != 0)
                    def _one(gi=gi, l=l, kp=kp):
                        dp = gi * L + l
                        for j in range(D // L):
                            sl = pl.ds(j * L, L)
                            rows_v[kp, sl] = rows_v[kp, sl] + rows_v[dp, sl]
                        kg = (kp // L) * L
                        uvec = uex_v[pl.ds(kg, L)]
                        udp = uvals[gi][l]
                        uex_v[pl.ds(kg, L)] = uvec + jnp.where(
                            iota16 == kp - kg, udp, 0.0)

            didx_v[pl.ds(gi * L, L)] = jnp.where(m_comb, TRASH + pos16, d16)

        for gi in range(NG):
            d16n = didx_v[pl.ds(gi * L, L)]
            u16 = uex_v[pl.ds(gi * L, L)]
            plsc.addupdate_scatter(den_v, [d16n], u16)

        pltpu.sync_copy(rows_v, u_sh.at[didx_v], add=True)
        return ()

    lax.fori_loop(0, NCHUNK, chunk_body, ())
    plsc.subcore_barrier()

    pltpu.sync_copy(den_v.at[pl.ds(0, N)], d_out.at[pl.ds(g * N, N)])
    for k in range(ROWS_PT // K):
        pltpu.sync_copy(u_sh.at[pl.ds(tb + k * K, K)], rows_v)
        pltpu.sync_copy(rows_v, u_out.at[core, pl.ds(tb + k * K, K)])


def _tc_final(h, u_parts, d_parts):
    B = 1000

    def body(h_ref, u_ref, d_ref, o_ref):
        h = h_ref[...]
        u = u_ref[0] + u_ref[1]
        dsum = jnp.sum(d_ref[...], axis=1)
        ind = dsum > 0.0
        inv = jnp.where(ind, 1.0 / jnp.where(ind, dsum, 1.0), 0.0)
        hd = jnp.where(ind[:, None], h - u * inv[:, None], 0.0)
        o_ref[...] = jnp.maximum(h + hd, 0.0)

    return pl.pallas_call(
        body,
        grid=(N // B,),
        in_specs=[
            pl.BlockSpec((B, D), lambda i: (i, 0)),
            pl.BlockSpec((NC, B, D), lambda i: (0, i, 0)),
            pl.BlockSpec((B, NW), lambda i: (i, 0)),
        ],
        out_specs=pl.BlockSpec((B, D), lambda i: (i, 0)),
        out_shape=jax.ShapeDtypeStruct((N, D), jnp.float32),
    )(h, u_parts, d_parts)


def kernel(h_init, edge_index, W1, Wa):
    h, p = _tc_fwd(h_init, W1.T, Wa.T)
    src = edge_index[0]
    dst = edge_index[1]
    u_parts, d_flat = _sc_edges(h, p.reshape(N), src, dst)
    return _tc_final(h, u_parts, d_flat.reshape(NW, N).T)
